# Initial kernel scaffold; baseline (speedup 1.0000x reference)
import jax
import jax.numpy as jnp
from jax.experimental import pallas as pl


def kernel(x, edge_index, batch, W1, b1, W2, b2, W3, b3, Wl, bl):
    raise NotImplementedError("write your pallas kernel here")


# speedup vs baseline: 14.1439x; 14.1439x over previous
import functools

import jax
import jax.numpy as jnp
from jax import lax
from jax.experimental import pallas as pl
from jax.experimental.pallas import tpu as pltpu
from jax.experimental.pallas import tpu_sc as plsc

N = 10000
E = 320000
D = 128
G = 64

NC = 2
NS = 16
EC = E // NC
ET = EC // NS
CH = 128
NFULL = ET // CH
TAIL = ET - NFULL * CH
NPAD = 10240
RPT = NPAD // NS
DEGW = 16

_mesh = plsc.VectorSubcoreMesh(core_axis_name="c", subcore_axis_name="s")


def _zero_vmem(ref, rows, width):
    def body(i, _):
        for j in range(width // 16):
            ref[i, pl.ds(j * 16, 16)] = jnp.zeros((16,), jnp.float32)
        return 0
    lax.fori_loop(0, rows, body, 0)


@functools.partial(
    pl.kernel,
    out_type=jax.ShapeDtypeStruct((NC, NPAD, DEGW), jnp.float32),
    mesh=_mesh,
    scratch_types=[
        pltpu.VMEM((CH,), jnp.int32),
        pltpu.VMEM((TAIL,), jnp.int32),
        pltpu.VMEM((CH, DEGW), jnp.float32),
        pltpu.VMEM((16, DEGW), jnp.float32),
        pltpu.VMEM_SHARED((NPAD, DEGW), jnp.float32),
    ],
)
def _sc_degree(dst_hbm, out_hbm, idx_d, idx_t, ones_v, zbuf, acc):
    c = lax.axis_index("c")
    s = lax.axis_index("s")

    def fill_ones(i, _):
        ones_v[i, pl.ds(0, 16)] = jnp.ones((16,), jnp.float32)
        return 0
    lax.fori_loop(0, CH, fill_ones, 0)
    _zero_vmem(zbuf, 16, DEGW)

    row0 = s * RPT
    def zinit(k, _):
        pltpu.sync_copy(zbuf, acc.at[pl.ds(row0 + k * 16, 16)])
        return 0
    lax.fori_loop(0, RPT // 16, zinit, 0)
    plsc.subcore_barrier()

    base = c * EC + s * ET
    def chunk(i, _):
        pltpu.sync_copy(dst_hbm.at[pl.ds(base + i * CH, CH)], idx_d)
        pltpu.sync_copy(ones_v, acc.at[idx_d], add=True)
        return 0
    lax.fori_loop(0, NFULL, chunk, 0)
    pltpu.sync_copy(dst_hbm.at[pl.ds(base + NFULL * CH, TAIL)], idx_t)
    pltpu.sync_copy(ones_v.at[pl.ds(0, TAIL)], acc.at[idx_t], add=True)

    plsc.subcore_barrier()
    pltpu.sync_copy(acc.at[pl.ds(row0, RPT)], out_hbm.at[c, pl.ds(row0, RPT)])


@functools.partial(
    pl.kernel,
    out_type=jax.ShapeDtypeStruct((NC, NPAD, D), jnp.float32),
    mesh=_mesh,
    scratch_types=[
        pltpu.VMEM((CH,), jnp.int32),
        pltpu.VMEM((TAIL,), jnp.int32),
        pltpu.VMEM((CH, D), jnp.float32),
        pltpu.VMEM((TAIL, D), jnp.float32),
        pltpu.VMEM((16, D), jnp.float32),
        pltpu.VMEM_SHARED((NPAD, D), jnp.float32),
        pltpu.SemaphoreType.DMA,
    ],
)
def _sc_aggregate(h_hbm, src_hbm, dst_hbm, out_hbm,
                  idx_s, idx_t, rows, rows_t, zbuf, acc, sem):
    c = lax.axis_index("c")
    s = lax.axis_index("s")

    _zero_vmem(zbuf, 16, D)
    row0 = s * RPT
    def zinit(k, _):
        pltpu.sync_copy(zbuf, acc.at[pl.ds(row0 + k * 16, 16)])
        return 0
    lax.fori_loop(0, RPT // 16, zinit, 0)
    plsc.subcore_barrier()

    base = c * EC + s * ET
    def chunk(i, _):
        off = base + i * CH
        pltpu.sync_copy(src_hbm.at[pl.ds(off, CH)], idx_s)
        pltpu.async_copy(h_hbm.at[idx_s], rows, sem).wait()
        pltpu.sync_copy(dst_hbm.at[pl.ds(off, CH)], idx_s)
        pltpu.sync_copy(rows, acc.at[idx_s], add=True)
        return 0
    lax.fori_loop(0, NFULL, chunk, 0)
    off = base + NFULL * CH
    pltpu.sync_copy(src_hbm.at[pl.ds(off, TAIL)], idx_t)
    pltpu.async_copy(h_hbm.at[idx_t], rows_t, sem).wait()
    pltpu.sync_copy(dst_hbm.at[pl.ds(off, TAIL)], idx_t)
    pltpu.sync_copy(rows_t, acc.at[idx_t], add=True)

    plsc.subcore_barrier()
    pltpu.sync_copy(acc.at[pl.ds(row0, RPT)], out_hbm.at[c, pl.ds(row0, RPT)])


def _mm_body(x_ref, w_ref, o_ref):
    o_ref[...] = jnp.dot(x_ref[...], w_ref[...],
                         preferred_element_type=jnp.float32)


def _tc_matmul(x, w):
    return pl.pallas_call(
        _mm_body,
        out_shape=jax.ShapeDtypeStruct((x.shape[0], w.shape[1]), jnp.float32),
    )(x, w)


def _scale_body(dp_ref, xw_ref, dis_ref, h1p_ref):
    deg = dp_ref[0, 0:N, 0:1] + dp_ref[1, 0:N, 0:1] + 1.0
    dis = lax.rsqrt(deg)
    dis_ref[...] = dis
    h1p_ref[...] = xw_ref[...] * dis


def _tc_scale(deg_partials, xw):
    return pl.pallas_call(
        _scale_body,
        out_shape=[
            jax.ShapeDtypeStruct((N, 1), jnp.float32),
            jax.ShapeDtypeStruct((N, D), jnp.float32),
        ],
    )(deg_partials, xw)


def _layer_body(ap_ref, hp_ref, dis_ref, b_ref, w_ref, o_ref):
    dis = dis_ref[...]
    t = (ap_ref[0, 0:N] + ap_ref[1, 0:N] + hp_ref[...]) * dis + b_ref[...]
    h = jnp.maximum(t, 0.0)
    o_ref[...] = jnp.dot(h, w_ref[...], preferred_element_type=jnp.float32) * dis


def _tc_layer(agg_partials, hp, dis, b2d, w):
    return pl.pallas_call(
        _layer_body,
        out_shape=jax.ShapeDtypeStruct((N, D), jnp.float32),
    )(agg_partials, hp, dis, b2d, w)


def _final_body(ap_ref, hp_ref, dis_ref, b_ref, batch_ref, wl_ref, bl_ref, o_ref):
    h3 = (ap_ref[0, 0:N] + ap_ref[1, 0:N] + hp_ref[...]) * dis_ref[...] + b_ref[...]
    gids = lax.broadcasted_iota(jnp.int32, (N, G), 1)
    onehot = (batch_ref[...] == gids).astype(jnp.float32)
    pooled = lax.dot_general(onehot, h3, (((0,), (0,)), ((), ())),
                             preferred_element_type=jnp.float32)
    cnt = jnp.sum(onehot, axis=0)[:, None]
    g = pooled / jnp.maximum(cnt, 1.0)
    z = jnp.dot(g, wl_ref[...], preferred_element_type=jnp.float32) + bl_ref[...]
    o_ref[...] = 1.0 / (1.0 + jnp.exp(-z))


def _tc_final(agg_partials, hp, dis, b2d, batch2d, wl, bl2d):
    return pl.pallas_call(
        _final_body,
        out_shape=jax.ShapeDtypeStruct((G, 1), jnp.float32),
    )(agg_partials, hp, dis, b2d, batch2d, wl, bl2d)


def kernel(x, edge_index, batch, W1, b1, W2, b2, W3, b3, Wl, bl):
    src = edge_index[0]
    dst = edge_index[1]
    batch2d = batch.reshape(N, 1)

    deg_partials = _sc_degree(dst)
    xw = _tc_matmul(x, W1)
    dis, hp = _tc_scale(deg_partials, xw)

    agg = _sc_aggregate(hp, src, dst)
    hp = _tc_layer(agg, hp, dis, b1.reshape(1, D), W2)
    agg = _sc_aggregate(hp, src, dst)
    hp = _tc_layer(agg, hp, dis, b2.reshape(1, D), W3)
    agg = _sc_aggregate(hp, src, dst)
    return _tc_final(agg, hp, dis, b3.reshape(1, D), batch2d, Wl,
                     bl.reshape(1, 1))
